# baseline (speedup 1.0000x reference)
import functools

import jax
import jax.numpy as jnp
from jax import lax
from jax.experimental import pallas as pl
from jax.experimental.pallas import tpu as pltpu
from jax.experimental.pallas import tpu_sc as plsc

NUM_CLASSES = 150
QPC = 5
KPC = 6
D = 256
B = 32
Q = NUM_CLASSES * QPC
NCOL = NUM_CLASSES + 1
LANES = 16
DCH = D // LANES
CPW = 5
NW = NUM_CLASSES // CPW
QPW = CPW * QPC
EPW = CPW * KPC


def _sc_similarities(query_features, emb_table):
    mesh = plsc.VectorSubcoreMesh(
        core_axis_name="c", subcore_axis_name="s", num_cores=2, num_subcores=16
    )

    @functools.partial(
        pl.kernel,
        out_type=jax.ShapeDtypeStruct((Q, 2, B), jnp.float32),
        mesh=mesh,
        scratch_types=[
            pltpu.VMEM((EPW, D), jnp.float32),
            pltpu.VMEM((QPW, D), jnp.float32),
            pltpu.VMEM((QPW, D), jnp.float32),
            pltpu.VMEM((QPW, 2, B), jnp.float32),
            pltpu.VMEM((QPW,), jnp.int32),
            pltpu.VMEM((EPW,), jnp.int32),
            pltpu.SemaphoreType.DMA,
            pltpu.SemaphoreType.DMA,
        ],
        compiler_params=pltpu.CompilerParams(needs_layout_passes=False),
    )
    def sc_kernel(q_hbm, e_hbm, out_hbm, e_v, q_v0, q_v1, o_v, qi_v, ei_v,
                  sem0, sem1):
        wid = lax.axis_index("s") * 2 + lax.axis_index("c")
        iota = lax.iota(jnp.int32, LANES)

        @pl.when(wid < NW)
        def _():
            qlo = wid * QPW
            qi_v[pl.ds(0, LANES)] = qlo + iota
            qi_v[pl.ds(QPW - LANES, LANES)] = qlo + (QPW - LANES) + iota
            ei_v[pl.ds(0, LANES)] = wid * EPW + iota
            ei_v[pl.ds(EPW - LANES, LANES)] = wid * EPW + (EPW - LANES) + iota

            e_cp = pltpu.async_copy(e_hbm.at[ei_v], e_v, sem1)
            pltpu.async_copy(q_hbm.at[0].at[qi_v], q_v0, sem0)
            e_cp.wait()

            zeros_i = iota * 0
            ones_i = zeros_i + 1
            lane15 = iota == LANES - 1
            def compute(b, q_v):
                b_lane = zeros_i + b
                for ci in range(CPW):
                    for j0, j1 in ((0, 3), (3, QPC)):
                        acc = [
                            [jnp.zeros((LANES,), jnp.float32)
                             for _ in range(KPC)]
                            for _ in range(j1 - j0)
                        ]
                        for dch in range(DCH):
                            sl = pl.ds(dch * LANES, LANES)
                            ev = [e_v[ci * KPC + k, sl] for k in range(KPC)]
                            for jj in range(j0, j1):
                                qv = q_v[ci * QPC + jj, sl]
                                for k in range(KPC):
                                    acc[jj - j0][k] = (
                                        acc[jj - j0][k] + qv * ev[k]
                                    )
                        for jj in range(j0, j1):
                            a = acc[jj - j0]
                            cums = [plsc.cumsum(a[k]) for k in range(KPC)]
                            pos01 = jnp.maximum(cums[0], cums[1])
                            pos23 = jnp.maximum(cums[2], cums[3])
                            pos = jnp.maximum(
                                jnp.maximum(pos01, pos23), cums[4]
                            )
                            row_b = zeros_i + (ci * QPC + jj)
                            plsc.store_scatter(
                                o_v, [row_b, zeros_i, b_lane], pos,
                                mask=lane15,
                            )
                            plsc.store_scatter(
                                o_v, [row_b, ones_i, b_lane], cums[KPC - 1],
                                mask=lane15,
                            )

            def body_even(b):
                pltpu.async_copy(q_hbm.at[b + 1].at[qi_v], q_v1, sem1)
                pltpu.make_async_copy(q_hbm.at[b].at[qi_v], q_v0, sem0).wait()
                compute(b, q_v0)

            def body_odd(b):
                pltpu.async_copy(
                    q_hbm.at[lax.rem(b + 1, B)].at[qi_v], q_v0, sem0
                )
                pltpu.make_async_copy(q_hbm.at[b].at[qi_v], q_v1, sem1).wait()
                compute(b, q_v1)

            def body(b, carry):
                lax.cond(lax.rem(b, 2) == 0, body_even, body_odd, b)
                return carry

            lax.fori_loop(0, B, body, 0)
            pltpu.make_async_copy(q_hbm.at[0].at[qi_v], q_v0, sem0).wait()
            pltpu.sync_copy(o_v, out_hbm.at[pl.ds(qlo, QPW)])

    return sc_kernel(query_features, emb_table)


def _tc_materialize(compact):
    def tc_body(c_ref, o_ref):
        row = lax.broadcasted_iota(jnp.int32, (Q, NCOL), 0)
        col = lax.broadcasted_iota(jnp.int32, (Q, NCOL), 1)
        mask_pos = col == row // QPC
        mask_bg = col == NCOL - 1
        ninf = jnp.float32(-jnp.inf)
        for b in range(B):
            pos = c_ref[b, :, 0:1]
            bg = c_ref[b, :, 1:2]
            o_ref[b] = jnp.where(mask_pos, pos, jnp.where(mask_bg, bg, ninf))

    return pl.pallas_call(
        tc_body,
        out_shape=jax.ShapeDtypeStruct((B, Q, NCOL), jnp.float32),
    )(compact)


def kernel(query_features, emb_table):
    compact = _sc_similarities(query_features, emb_table)
    return _tc_materialize(jnp.transpose(compact, (2, 0, 1)))


# speedup vs baseline: 1.0640x; 1.0640x over previous
import functools

import jax
import jax.numpy as jnp
from jax import lax
from jax.experimental import pallas as pl
from jax.experimental.pallas import tpu as pltpu
from jax.experimental.pallas import tpu_sc as plsc

NUM_CLASSES = 150
QPC = 5
KPC = 6
D = 256
B = 32
Q = NUM_CLASSES * QPC
NCOL = NUM_CLASSES + 1
LANES = 16
DCH = D // LANES
CPW = 5
NW = NUM_CLASSES // CPW
QPW = CPW * QPC
EPW = CPW * KPC


def _sc_similarities(query_features, emb_table):
    mesh = plsc.VectorSubcoreMesh(
        core_axis_name="c", subcore_axis_name="s", num_cores=2, num_subcores=16
    )

    @functools.partial(
        pl.kernel,
        out_type=jax.ShapeDtypeStruct((Q, 2, B), jnp.float32),
        mesh=mesh,
        scratch_types=[
            pltpu.VMEM((EPW, D), jnp.float32),
            pltpu.VMEM((QPW, D), jnp.float32),
            pltpu.VMEM((QPW, D), jnp.float32),
            pltpu.VMEM((QPW, 2, B), jnp.float32),
            pltpu.VMEM((QPW,), jnp.int32),
            pltpu.VMEM((EPW,), jnp.int32),
            pltpu.SemaphoreType.DMA,
            pltpu.SemaphoreType.DMA,
        ],
        compiler_params=pltpu.CompilerParams(needs_layout_passes=False),
    )
    def sc_kernel(q_hbm, e_hbm, out_hbm, e_v, q_v0, q_v1, o_v, qi_v, ei_v,
                  sem0, sem1):
        wid = lax.axis_index("s") * 2 + lax.axis_index("c")
        iota = lax.iota(jnp.int32, LANES)

        @pl.when(wid < NW)
        def _():
            qlo = wid * QPW
            qi_v[pl.ds(0, LANES)] = qlo + iota
            qi_v[pl.ds(QPW - LANES, LANES)] = qlo + (QPW - LANES) + iota
            ei_v[pl.ds(0, LANES)] = wid * EPW + iota
            ei_v[pl.ds(EPW - LANES, LANES)] = wid * EPW + (EPW - LANES) + iota

            e_cp = pltpu.async_copy(e_hbm.at[ei_v], e_v, sem1)
            pltpu.async_copy(q_hbm.at[0].at[qi_v], q_v0, sem0)
            e_cp.wait()

            zeros_i = iota * 0
            ones_i = zeros_i + 1
            lane15 = iota == LANES - 1
            def compute(b, q_v):
                b_lane = zeros_i + b
                for ci in range(CPW):
                    for j0, j1 in ((0, QPC),):
                        acc = [
                            [jnp.zeros((LANES,), jnp.float32)
                             for _ in range(KPC)]
                            for _ in range(j1 - j0)
                        ]
                        for dch in range(DCH):
                            sl = pl.ds(dch * LANES, LANES)
                            ev = [e_v[ci * KPC + k, sl] for k in range(KPC)]
                            for jj in range(j0, j1):
                                qv = q_v[ci * QPC + jj, sl]
                                for k in range(KPC):
                                    acc[jj - j0][k] = (
                                        acc[jj - j0][k] + qv * ev[k]
                                    )
                        for jj in range(j0, j1):
                            a = acc[jj - j0]
                            cums = [plsc.cumsum(a[k]) for k in range(KPC)]
                            pos01 = jnp.maximum(cums[0], cums[1])
                            pos23 = jnp.maximum(cums[2], cums[3])
                            pos = jnp.maximum(
                                jnp.maximum(pos01, pos23), cums[4]
                            )
                            row_b = zeros_i + (ci * QPC + jj)
                            plsc.store_scatter(
                                o_v, [row_b, zeros_i, b_lane], pos,
                                mask=lane15,
                            )
                            plsc.store_scatter(
                                o_v, [row_b, ones_i, b_lane], cums[KPC - 1],
                                mask=lane15,
                            )

            def body(i, carry):
                b0 = 2 * i
                b1 = 2 * i + 1
                pltpu.async_copy(q_hbm.at[b1].at[qi_v], q_v1, sem1)
                pltpu.make_async_copy(q_hbm.at[b0].at[qi_v], q_v0, sem0).wait()
                compute(b0, q_v0)
                pltpu.async_copy(
                    q_hbm.at[lax.rem(b0 + 2, B)].at[qi_v], q_v0, sem0
                )
                pltpu.make_async_copy(q_hbm.at[b1].at[qi_v], q_v1, sem1).wait()
                compute(b1, q_v1)
                return carry

            lax.fori_loop(0, B // 2, body, 0)
            pltpu.make_async_copy(q_hbm.at[0].at[qi_v], q_v0, sem0).wait()
            pltpu.sync_copy(o_v, out_hbm.at[pl.ds(qlo, QPW)])

    return sc_kernel(query_features, emb_table)


def _tc_materialize(compact):
    def tc_body(c_ref, o_ref):
        row = lax.broadcasted_iota(jnp.int32, (Q, NCOL), 0)
        col = lax.broadcasted_iota(jnp.int32, (Q, NCOL), 1)
        mask_pos = col == row // QPC
        mask_bg = col == NCOL - 1
        ninf = jnp.float32(-jnp.inf)
        for b in range(B):
            pos = c_ref[b, :, 0:1]
            bg = c_ref[b, :, 1:2]
            o_ref[b] = jnp.where(mask_pos, pos, jnp.where(mask_bg, bg, ninf))

    return pl.pallas_call(
        tc_body,
        out_shape=jax.ShapeDtypeStruct((B, Q, NCOL), jnp.float32),
    )(compact)


def kernel(query_features, emb_table):
    compact = _sc_similarities(query_features, emb_table)
    return _tc_materialize(jnp.transpose(compact, (2, 0, 1)))


# speedup vs baseline: 1.3561x; 1.2745x over previous
import functools

import jax
import jax.numpy as jnp
from jax import lax
from jax.experimental import pallas as pl
from jax.experimental.pallas import tpu as pltpu
from jax.experimental.pallas import tpu_sc as plsc

NUM_CLASSES = 150
QPC = 5
KPC = 6
D = 256
B = 32
Q = NUM_CLASSES * QPC
NCOL = NUM_CLASSES + 1
LANES = 16
DCH = D // LANES

SC_CLASSES = 90
CPW = 3
NW = SC_CLASSES // CPW
QPW = CPW * QPC
EPW = CPW * KPC
SC_Q = SC_CLASSES * QPC

TC_C0 = 88
TC_NC = NUM_CLASSES - TC_C0
TC_Q = TC_NC * QPC
TC_E = TC_NC * KPC


def _sc_similarities(query_features, emb_table):
    mesh = plsc.VectorSubcoreMesh(
        core_axis_name="c", subcore_axis_name="s", num_cores=2, num_subcores=16
    )

    @functools.partial(
        pl.kernel,
        out_type=jax.ShapeDtypeStruct((SC_Q, 2, B), jnp.float32),
        mesh=mesh,
        scratch_types=[
            pltpu.VMEM((EPW, D), jnp.float32),
            pltpu.VMEM((LANES, D), jnp.float32),
            pltpu.VMEM((LANES, D), jnp.float32),
            pltpu.VMEM((QPW, 2, B), jnp.float32),
            pltpu.VMEM((LANES,), jnp.int32),
            pltpu.VMEM((EPW,), jnp.int32),
            pltpu.SemaphoreType.DMA,
            pltpu.SemaphoreType.DMA,
        ],
        compiler_params=pltpu.CompilerParams(needs_layout_passes=False),
    )
    def sc_kernel(q_hbm, e_hbm, out_hbm, e_v, q_v0, q_v1, o_v, qi_v, ei_v,
                  sem0, sem1):
        wid = lax.axis_index("s") * 2 + lax.axis_index("c")
        iota = lax.iota(jnp.int32, LANES)

        @pl.when(wid < NW)
        def _():
            qlo = wid * QPW
            qi_v[pl.ds(0, LANES)] = qlo + iota
            ei_v[pl.ds(0, LANES)] = wid * EPW + iota
            ei_v[pl.ds(EPW - LANES, LANES)] = wid * EPW + (EPW - LANES) + iota

            e_cp = pltpu.async_copy(e_hbm.at[ei_v], e_v, sem1)
            pltpu.async_copy(q_hbm.at[0].at[qi_v], q_v0, sem0)
            e_cp.wait()

            zeros_i = iota * 0
            ones_i = zeros_i + 1
            lane15 = iota == LANES - 1

            def compute(b, q_v):
                b_lane = zeros_i + b
                for ci in range(CPW):
                    acc = [
                        [jnp.zeros((LANES,), jnp.float32) for _ in range(KPC)]
                        for _ in range(QPC)
                    ]
                    for dch in range(DCH):
                        sl = pl.ds(dch * LANES, LANES)
                        ev = [e_v[ci * KPC + k, sl] for k in range(KPC)]
                        for jj in range(QPC):
                            qv = q_v[ci * QPC + jj, sl]
                            for k in range(KPC):
                                acc[jj][k] = acc[jj][k] + qv * ev[k]
                    for jj in range(QPC):
                        a = acc[jj]
                        cums = [plsc.cumsum(a[k]) for k in range(KPC)]
                        pos01 = jnp.maximum(cums[0], cums[1])
                        pos23 = jnp.maximum(cums[2], cums[3])
                        pos = jnp.maximum(jnp.maximum(pos01, pos23), cums[4])
                        row_b = zeros_i + (ci * QPC + jj)
                        plsc.store_scatter(
                            o_v, [row_b, zeros_i, b_lane], pos, mask=lane15
                        )
                        plsc.store_scatter(
                            o_v, [row_b, ones_i, b_lane], cums[KPC - 1],
                            mask=lane15,
                        )

            def body(i, carry):
                b0 = 2 * i
                b1 = 2 * i + 1
                pltpu.async_copy(q_hbm.at[b1].at[qi_v], q_v1, sem1)
                pltpu.make_async_copy(q_hbm.at[b0].at[qi_v], q_v0, sem0).wait()
                compute(b0, q_v0)
                pltpu.async_copy(
                    q_hbm.at[lax.rem(b0 + 2, B)].at[qi_v], q_v0, sem0
                )
                pltpu.make_async_copy(q_hbm.at[b1].at[qi_v], q_v1, sem1).wait()
                compute(b1, q_v1)
                return carry

            lax.fori_loop(0, B // 2, body, 0)
            pltpu.make_async_copy(q_hbm.at[0].at[qi_v], q_v0, sem0).wait()
            pltpu.sync_copy(o_v, out_hbm.at[pl.ds(qlo, QPW)])

    return sc_kernel(query_features, emb_table)


def _tc_sims(query_features, emb_table):
    def body(q_ref, e_ref, o_ref):
        q = q_ref[0, TC_C0 * QPC:, :]
        e = e_ref[TC_C0 * KPC:, :]
        sims = lax.dot_general(
            q, e, (((1,), (1,)), ((), ())),
            preferred_element_type=jnp.float32,
        )
        row = lax.broadcasted_iota(jnp.int32, (TC_Q, TC_E), 0)
        col = lax.broadcasted_iota(jnp.int32, (TC_Q, TC_E), 1)
        same = col // KPC == row // QPC
        k = col - (col // KPC) * KPC
        ninf = jnp.float32(-jnp.inf)
        pos = jnp.max(jnp.where(same & (k < QPC), sims, ninf), axis=1)
        bg = jnp.max(jnp.where(same & (k == QPC), sims, ninf), axis=1)
        o_ref[0] = jnp.stack([pos, bg], axis=1)

    return pl.pallas_call(
        body,
        grid=(B,),
        in_specs=[
            pl.BlockSpec((1, Q, D), lambda b: (b, 0, 0)),
            pl.BlockSpec((NUM_CLASSES * KPC, D), lambda b: (0, 0)),
        ],
        out_specs=pl.BlockSpec((1, TC_Q, 2), lambda b: (b, 0, 0)),
        out_shape=jax.ShapeDtypeStruct((B, TC_Q, 2), jnp.float32),
    )(query_features, emb_table)


def _tc_materialize(compact):
    def tc_body(c_ref, o_ref):
        row = lax.broadcasted_iota(jnp.int32, (Q, NCOL), 0)
        col = lax.broadcasted_iota(jnp.int32, (Q, NCOL), 1)
        mask_pos = col == row // QPC
        mask_bg = col == NCOL - 1
        ninf = jnp.float32(-jnp.inf)
        for b in range(B):
            pos = c_ref[b, :, 0:1]
            bg = c_ref[b, :, 1:2]
            o_ref[b] = jnp.where(mask_pos, pos, jnp.where(mask_bg, bg, ninf))

    return pl.pallas_call(
        tc_body,
        out_shape=jax.ShapeDtypeStruct((B, Q, NCOL), jnp.float32),
    )(compact)


def kernel(query_features, emb_table):
    sc_c = _sc_similarities(query_features, emb_table)
    tc_c = _tc_sims(query_features, emb_table)
    compact = jnp.concatenate(
        [jnp.transpose(sc_c, (2, 0, 1)),
         tc_c[:, (SC_CLASSES - TC_C0) * QPC:, :]],
        axis=1,
    )
    return _tc_materialize(compact)


# speedup vs baseline: 1.7202x; 1.2686x over previous
import functools

import jax
import jax.numpy as jnp
from jax import lax
from jax.experimental import pallas as pl
from jax.experimental.pallas import tpu as pltpu
from jax.experimental.pallas import tpu_sc as plsc

NUM_CLASSES = 150
QPC = 5
KPC = 6
D = 256
B = 32
Q = NUM_CLASSES * QPC
NCOL = NUM_CLASSES + 1
LANES = 16
DCH = D // LANES

SC_CLASSES = 64
CPW = 2
NW = SC_CLASSES // CPW
QPW = CPW * QPC
EPW = CPW * KPC
SC_Q = SC_CLASSES * QPC

TC_C0 = SC_CLASSES
TC_NC = NUM_CLASSES - TC_C0
TC_Q = TC_NC * QPC
TC_E = TC_NC * KPC


def _sc_similarities(query_features, emb_table):
    mesh = plsc.VectorSubcoreMesh(
        core_axis_name="c", subcore_axis_name="s", num_cores=2, num_subcores=16
    )

    @functools.partial(
        pl.kernel,
        out_type=jax.ShapeDtypeStruct((SC_Q, 2, B), jnp.float32),
        mesh=mesh,
        scratch_types=[
            pltpu.VMEM((LANES, D), jnp.float32),
            pltpu.VMEM((LANES, D), jnp.float32),
            pltpu.VMEM((LANES, D), jnp.float32),
            pltpu.VMEM((QPW, 2, B), jnp.float32),
            pltpu.VMEM((LANES,), jnp.int32),
            pltpu.VMEM((LANES,), jnp.int32),
            pltpu.SemaphoreType.DMA,
            pltpu.SemaphoreType.DMA,
        ],
        compiler_params=pltpu.CompilerParams(needs_layout_passes=False),
    )
    def sc_kernel(q_hbm, e_hbm, out_hbm, e_v, q_v0, q_v1, o_v, qi_v, ei_v,
                  sem0, sem1):
        wid = lax.axis_index("s") * 2 + lax.axis_index("c")
        iota = lax.iota(jnp.int32, LANES)

        @pl.when(wid < NW)
        def _():
            qlo = wid * QPW
            qi_v[pl.ds(0, LANES)] = qlo + iota
            ei_v[pl.ds(0, LANES)] = wid * EPW + iota

            e_cp = pltpu.async_copy(e_hbm.at[ei_v], e_v, sem1)
            pltpu.async_copy(q_hbm.at[0].at[qi_v], q_v0, sem0)
            e_cp.wait()

            zeros_i = iota * 0
            ones_i = zeros_i + 1
            lane15 = iota == LANES - 1

            def compute(b, q_v):
                b_lane = zeros_i + b
                for ci in range(CPW):
                    acc = [
                        [jnp.zeros((LANES,), jnp.float32) for _ in range(KPC)]
                        for _ in range(QPC)
                    ]
                    for dch in range(DCH):
                        sl = pl.ds(dch * LANES, LANES)
                        ev = [e_v[ci * KPC + k, sl] for k in range(KPC)]
                        for jj in range(QPC):
                            qv = q_v[ci * QPC + jj, sl]
                            for k in range(KPC):
                                acc[jj][k] = acc[jj][k] + qv * ev[k]
                    for jj in range(QPC):
                        a = acc[jj]
                        cums = [plsc.cumsum(a[k]) for k in range(KPC)]
                        pos01 = jnp.maximum(cums[0], cums[1])
                        pos23 = jnp.maximum(cums[2], cums[3])
                        pos = jnp.maximum(jnp.maximum(pos01, pos23), cums[4])
                        row_b = zeros_i + (ci * QPC + jj)
                        plsc.store_scatter(
                            o_v, [row_b, zeros_i, b_lane], pos, mask=lane15
                        )
                        plsc.store_scatter(
                            o_v, [row_b, ones_i, b_lane], cums[KPC - 1],
                            mask=lane15,
                        )

            def body(i, carry):
                b0 = 2 * i
                b1 = 2 * i + 1
                pltpu.async_copy(q_hbm.at[b1].at[qi_v], q_v1, sem1)
                pltpu.make_async_copy(q_hbm.at[b0].at[qi_v], q_v0, sem0).wait()
                compute(b0, q_v0)
                pltpu.async_copy(
                    q_hbm.at[lax.rem(b0 + 2, B)].at[qi_v], q_v0, sem0
                )
                pltpu.make_async_copy(q_hbm.at[b1].at[qi_v], q_v1, sem1).wait()
                compute(b1, q_v1)
                return carry

            lax.fori_loop(0, B // 2, body, 0)
            pltpu.make_async_copy(q_hbm.at[0].at[qi_v], q_v0, sem0).wait()
            pltpu.sync_copy(o_v, out_hbm.at[pl.ds(qlo, QPW)])

    return sc_kernel(query_features, emb_table)


def _tc_sims(query_features, emb_table):
    def body(q_ref, e_ref, o_ref):
        q = q_ref[0, TC_C0 * QPC:, :].astype(jnp.bfloat16)
        e = e_ref[TC_C0 * KPC:, :].astype(jnp.bfloat16)
        sims = lax.dot_general(
            q, e, (((1,), (1,)), ((), ())),
            preferred_element_type=jnp.float32,
        )
        row = lax.broadcasted_iota(jnp.int32, (TC_Q, TC_E), 0)
        col = lax.broadcasted_iota(jnp.int32, (TC_Q, TC_E), 1)
        same = col // KPC == row // QPC
        k = col - (col // KPC) * KPC
        ninf = jnp.float32(-jnp.inf)
        pos = jnp.max(jnp.where(same & (k < QPC), sims, ninf), axis=1)
        bg = jnp.max(jnp.where(same & (k == QPC), sims, ninf), axis=1)
        o_ref[0] = jnp.stack([pos, bg], axis=1)

    return pl.pallas_call(
        body,
        grid=(B,),
        in_specs=[
            pl.BlockSpec((1, Q, D), lambda b: (b, 0, 0)),
            pl.BlockSpec((NUM_CLASSES * KPC, D), lambda b: (0, 0)),
        ],
        out_specs=pl.BlockSpec((1, TC_Q, 2), lambda b: (b, 0, 0)),
        out_shape=jax.ShapeDtypeStruct((B, TC_Q, 2), jnp.float32),
    )(query_features, emb_table)


def _tc_materialize(sc_t, tc_c):
    def masks(nrows, r0):
        row = r0 + lax.broadcasted_iota(jnp.int32, (nrows, NCOL), 0)
        col = lax.broadcasted_iota(jnp.int32, (nrows, NCOL), 1)
        return col == row // QPC, col == NCOL - 1

    def tc_body(c1_ref, c2_ref, o_ref):
        mp1, mb1 = masks(SC_Q, 0)
        mp2, mb2 = masks(Q - SC_Q, SC_Q)
        ninf = jnp.float32(-jnp.inf)
        for b in range(B):
            o_ref[b, pl.ds(0, SC_Q), :] = jnp.where(
                mp1, c1_ref[b, :, 0:1],
                jnp.where(mb1, c1_ref[b, :, 1:2], ninf),
            )
            o_ref[b, pl.ds(SC_Q, Q - SC_Q), :] = jnp.where(
                mp2, c2_ref[b, :, 0:1],
                jnp.where(mb2, c2_ref[b, :, 1:2], ninf),
            )

    return pl.pallas_call(
        tc_body,
        out_shape=jax.ShapeDtypeStruct((B, Q, NCOL), jnp.float32),
    )(sc_t, tc_c)


def kernel(query_features, emb_table):
    sc_c = _sc_similarities(query_features, emb_table)
    tc_c = _tc_sims(query_features, emb_table)
    return _tc_materialize(jnp.transpose(sc_c, (2, 0, 1)), tc_c)
